# baseline (speedup 1.0000x reference)
import functools

import jax
import jax.numpy as jnp
import numpy as np
from jax.experimental import pallas as pl
from jax.experimental.pallas import tpu as pltpu

_N_E = 10000
_N_M = 64
_D = 128
_LEVELS = 6
_R = 6
_NREL = _LEVELS - 1
_BLOCK = 2000


def _hetero_kernel(w_ref, xe_ref, xm_ref, wr1c_ref, wq1_ref, b1_ref,
                   wr2c_ref, wq2_ref, b2_ref, out_ref, h_ref, bd_ref):
    f32 = jnp.float32

    @pl.when(pl.program_id(0) == 0)
    def _build_h():
        xm = xm_ref[...]
        xm1 = jnp.maximum(
            jnp.dot(xm, wq1_ref[...], preferred_element_type=f32)
            + b1_ref[...], 0.0)
        t1 = jnp.dot(xm, wr1c_ref[...], preferred_element_type=f32)
        t2 = jnp.dot(xm1, wr2c_ref[...], preferred_element_type=f32)
        for r in range(_NREL):
            h_ref[r * _N_M:(r + 1) * _N_M, :_D] = t1[:, r * _D:(r + 1) * _D]
            h_ref[r * _N_M:(r + 1) * _N_M, _D:] = t2[:, r * _D:(r + 1) * _D]
        i = jax.lax.broadcasted_iota(jnp.int32, (_NREL * _N_M,) * 2, 0)
        j = jax.lax.broadcasted_iota(jnp.int32, (_NREL * _N_M,) * 2, 1)
        bd_ref[...] = ((i // _N_M) == (j // _N_M)).astype(f32)

    w = w_ref[...]
    xe = xe_ref[...]
    wq1 = wq1_ref[...]
    wq2 = wq2_ref[...]
    b1 = b1_ref[...]
    b2 = b2_ref[...]

    g = [(w > np.float32(r / _LEVELS)).astype(f32)
         for r in range(1, _LEVELS + 1)]
    m_all = jnp.concatenate(
        [g[k] - g[k + 1] for k in range(_NREL)], axis=1)

    cexp = jnp.dot(m_all, bd_ref[...], preferred_element_type=f32)
    p = m_all / jnp.maximum(cexp, 1.0)

    agg = jnp.dot(p, h_ref[...], preferred_element_type=f32)
    out1 = agg[:, :_D] + jnp.dot(xe, wq1, preferred_element_type=f32) + b1
    out1 = jnp.maximum(out1, 0.0)
    out_ref[...] = (agg[:, _D:]
                    + jnp.dot(out1, wq2, preferred_element_type=f32) + b2)


@functools.partial(jax.jit, static_argnames=("interpret",))
def _run(w, xe, xm, wr1c, wq1, b1, wr2c, wq2, b2, interpret=False):
    grid = (_N_E // _BLOCK,)
    full = lambda shape: pl.BlockSpec(shape, lambda i: (0,) * len(shape))
    out = pl.pallas_call(
        _hetero_kernel,
        grid=grid,
        in_specs=[
            pl.BlockSpec((_BLOCK, _N_M), lambda i: (i, 0)),
            pl.BlockSpec((_BLOCK, _D), lambda i: (i, 0)),
            full((_N_M, _D)),
            full((_D, _NREL * _D)),
            full((_D, _D)),
            full((1, _D)),
            full((_D, _NREL * _D)),
            full((_D, _D)),
            full((1, _D)),
        ],
        out_specs=pl.BlockSpec((_BLOCK, _D), lambda i: (i, 0)),
        out_shape=jax.ShapeDtypeStruct((_N_E, _D), jnp.float32),
        scratch_shapes=[pltpu.VMEM((_NREL * _N_M, 2 * _D), jnp.float32),
                        pltpu.VMEM((_NREL * _N_M, _NREL * _N_M),
                                   jnp.float32)],
        interpret=interpret,
    )(w, xe, xm, wr1c, wq1, b1, wr2c, wq2, b2)
    return out


def kernel(emb_entity, emb_mole, entity_mole_weights,
           W_rel1, W_root1, b1, W_rel2, W_root2, b2):
    wr1c = W_rel1[1:_LEVELS].transpose(1, 0, 2).reshape(_D, _NREL * _D)
    wr2c = W_rel2[1:_LEVELS].transpose(1, 0, 2).reshape(_D, _NREL * _D)
    out = _run(entity_mole_weights, emb_entity[0], emb_mole[0],
               wr1c, W_root1, b1.reshape(1, _D),
               wr2c, W_root2, b2.reshape(1, _D))
    return out[None]


# speedup vs baseline: 1.1201x; 1.1201x over previous
import functools

import jax
import jax.numpy as jnp
import numpy as np
from jax.experimental import pallas as pl
from jax.experimental.pallas import tpu as pltpu

_N_E = 10000
_N_M = 64
_D = 128
_LEVELS = 6
_R = 6
_NREL = _LEVELS - 1
_BLOCK = 2000


def _hetero_kernel(w_ref, xe_ref, xm_ref, wr1_ref, wq1_ref, b1_ref,
                   wr2_ref, wq2_ref, b2_ref, out_ref, h_ref, bd_ref):
    f32 = jnp.float32

    @pl.when(pl.program_id(0) == 0)
    def _build_h():
        xm = xm_ref[...]
        xm1 = jnp.maximum(
            jnp.dot(xm, wq1_ref[...], preferred_element_type=f32)
            + b1_ref[...], 0.0)
        for r in range(1, _LEVELS):
            row = (r - 1) * _N_M
            h_ref[row:row + _N_M, :_D] = jnp.dot(
                xm, wr1_ref[r], preferred_element_type=f32)
            h_ref[row:row + _N_M, _D:] = jnp.dot(
                xm1, wr2_ref[r], preferred_element_type=f32)
        i = jax.lax.broadcasted_iota(jnp.int32, (_NREL * _N_M,) * 2, 0)
        j = jax.lax.broadcasted_iota(jnp.int32, (_NREL * _N_M,) * 2, 1)
        bd_ref[...] = ((i // _N_M) == (j // _N_M)).astype(f32)

    w = w_ref[...]
    xe = xe_ref[...]
    wq1 = wq1_ref[...]
    wq2 = wq2_ref[...]
    b1 = b1_ref[...]
    b2 = b2_ref[...]

    g = [(w > np.float32(r / _LEVELS)).astype(f32)
         for r in range(1, _LEVELS + 1)]
    m_all = jnp.concatenate(
        [g[k] - g[k + 1] for k in range(_NREL)], axis=1)

    cexp = jnp.dot(m_all, bd_ref[...], preferred_element_type=f32)
    p = m_all / jnp.maximum(cexp, 1.0)

    agg = jnp.dot(p, h_ref[...], preferred_element_type=f32)
    out1 = agg[:, :_D] + jnp.dot(xe, wq1, preferred_element_type=f32) + b1
    out1 = jnp.maximum(out1, 0.0)
    out_ref[...] = (agg[:, _D:]
                    + jnp.dot(out1, wq2, preferred_element_type=f32) + b2)


@functools.partial(jax.jit, static_argnames=("interpret",))
def _run(w, xe, xm, wr1c, wq1, b1, wr2c, wq2, b2, interpret=False):
    grid = (_N_E // _BLOCK,)
    full = lambda shape: pl.BlockSpec(shape, lambda i: (0,) * len(shape))
    out = pl.pallas_call(
        _hetero_kernel,
        grid=grid,
        in_specs=[
            pl.BlockSpec((_BLOCK, _N_M), lambda i: (i, 0)),
            pl.BlockSpec((_BLOCK, _D), lambda i: (i, 0)),
            full((_N_M, _D)),
            full((_R, _D, _D)),
            full((_D, _D)),
            full((1, _D)),
            full((_R, _D, _D)),
            full((_D, _D)),
            full((1, _D)),
        ],
        out_specs=pl.BlockSpec((_BLOCK, _D), lambda i: (i, 0)),
        out_shape=jax.ShapeDtypeStruct((_N_E, _D), jnp.float32),
        scratch_shapes=[pltpu.VMEM((_NREL * _N_M, 2 * _D), jnp.float32),
                        pltpu.VMEM((_NREL * _N_M, _NREL * _N_M),
                                   jnp.float32)],
        interpret=interpret,
    )(w, xe, xm, wr1c, wq1, b1, wr2c, wq2, b2)
    return out


def kernel(emb_entity, emb_mole, entity_mole_weights,
           W_rel1, W_root1, b1, W_rel2, W_root2, b2):
    out = _run(entity_mole_weights, emb_entity[0], emb_mole[0],
               W_rel1, W_root1, b1.reshape(1, _D),
               W_rel2, W_root2, b2.reshape(1, _D))
    return out[None]
